# baseline (speedup 1.0000x reference)
import functools

import jax
import jax.numpy as jnp
from jax import lax
from jax.experimental import pallas as pl
from jax.experimental.pallas import tpu as pltpu
from jax.experimental.pallas import tpu_sc as plsc

TOKENS = 16384
D_MODEL = 4096
N_EXPERTS = 64
BT = 1024
NK = 2
KS = D_MODEL // NK

N_CHUNKS = 1
TOK_C = TOKENS // N_CHUNKS

NC = 2
NS = 16
NW = NC * NS
L = 16
CHUNK = TOK_C // NW
NG = CHUNK // L


WPB = BT // (TOKENS // NW)
HALF = (TOKENS // NW) // 2


def _scores_block(*refs):
    x_refs = refs[:NK]
    w_ref = refs[NK]
    out_ref = refs[NK + 1]
    acc = jnp.zeros((N_EXPERTS, BT), jnp.float32)
    for j in range(NK):
        acc += lax.dot_general(
            w_ref[:, j * KS:(j + 1) * KS], x_refs[j][...],
            dimension_numbers=(((1,), (1,)), ((), ())),
            preferred_element_type=jnp.float32,
        )
    for a in range(WPB):
        for b in range(2):
            s = (a * 2 + b) * HALF
            out_ref[a, b] = acc[:, s:s + HALF]


def _tc_scores(x, W, c):
    grid = (TOK_C // BT,)
    t0 = c * (TOK_C // BT)
    return pl.pallas_call(
        _scores_block,
        grid=grid,
        in_specs=[
            pl.BlockSpec((BT, KS),
                         functools.partial(lambda j, t: (t + t0, j), j))
            for j in range(NK)
        ] + [
            pl.BlockSpec((N_EXPERTS, D_MODEL), lambda t: (0, 0)),
        ],
        out_specs=pl.BlockSpec((WPB, 2, N_EXPERTS, HALF),
                               lambda t: (t, 0, 0, 0)),
        out_shape=jax.ShapeDtypeStruct((NW, 2, N_EXPERTS, HALF),
                                       jnp.float32),
    )(*([x] * NK), W)


def _top2_update(v, e, st):
    m1, i1, m2, i2 = st
    ev = jnp.full((L,), e, jnp.int32)
    gt1 = v > m1
    gt2 = v > m2
    m2n = jnp.where(gt1, m1, jnp.where(gt2, v, m2))
    i2n = jnp.where(gt1, i1, jnp.where(gt2, ev, i2))
    m1n = jnp.where(gt1, v, m1)
    i1n = jnp.where(gt1, ev, i1)
    return (m1n, i1n, m2n, i2n)


NI = 4


def _sc_top2_body(scores_hbm, i1_hbm, i2_hbm, v1_hbm, v2_hbm,
                  sbuf_a, sbuf_b, i1b, i2b, v1b, v2b, sem_a, sem_b, sem_o):
    wid = lax.axis_index("s") * NC + lax.axis_index("c")
    base = wid * CHUNK
    cp_a = pltpu.async_copy(scores_hbm.at[wid, 0], sbuf_a, sem_a)
    cp_b = pltpu.async_copy(scores_hbm.at[wid, 1], sbuf_b, sem_b)

    def make_quad(sbuf, out_base):
        def quad(g, _):
            offs = [g * (NI * L) + k * L for k in range(NI)]
            neg = jnp.full((L,), -jnp.inf, jnp.float32)
            zero = jnp.zeros((L,), jnp.int32)
            sts = [(neg, zero, neg, zero)] * NI
            for e in range(N_EXPERTS):
                sts = [_top2_update(sbuf[e, pl.ds(off, L)], e, st)
                       for off, st in zip(offs, sts)]
            for off, (m1, i1, m2, i2) in zip(offs, sts):
                e2 = jnp.exp(m2 - m1)
                d = 1.0 + e2
                o = out_base + off
                i1b[pl.ds(o, L)] = i1
                i2b[pl.ds(o, L)] = i2
                v1b[pl.ds(o, L)] = 1.0 / d
                v2b[pl.ds(o, L)] = e2 / d
            return 0
        return quad

    half_quads = HALF // (NI * L)
    cp_a.wait()
    lax.fori_loop(0, half_quads, make_quad(sbuf_a, 0), 0)
    cp_b.wait()
    lax.fori_loop(0, half_quads, make_quad(sbuf_b, HALF), 0)
    cps = [pltpu.async_copy(b, h.at[pl.ds(base, CHUNK)], sem_o)
           for b, h in ((i1b, i1_hbm), (i2b, i2_hbm),
                        (v1b, v1_hbm), (v2b, v2_hbm))]
    for cp in cps:
        cp.wait()


def _sc_top2(scores_T):
    mesh = plsc.VectorSubcoreMesh(core_axis_name="c", subcore_axis_name="s")
    f = functools.partial(
        pl.kernel, mesh=mesh,
        out_type=[
            jax.ShapeDtypeStruct((TOK_C,), jnp.int32),
            jax.ShapeDtypeStruct((TOK_C,), jnp.int32),
            jax.ShapeDtypeStruct((TOK_C,), jnp.float32),
            jax.ShapeDtypeStruct((TOK_C,), jnp.float32),
        ],
        scratch_types=[
            pltpu.VMEM((N_EXPERTS, HALF), jnp.float32),
            pltpu.VMEM((N_EXPERTS, HALF), jnp.float32),
            pltpu.VMEM((CHUNK,), jnp.int32),
            pltpu.VMEM((CHUNK,), jnp.int32),
            pltpu.VMEM((CHUNK,), jnp.float32),
            pltpu.VMEM((CHUNK,), jnp.float32),
            pltpu.SemaphoreType.DMA,
            pltpu.SemaphoreType.DMA,
            pltpu.SemaphoreType.DMA,
        ],
    )(_sc_top2_body)
    return f(scores_T)


@jax.jit
def kernel(x, W):
    outs = []
    for c in range(N_CHUNKS):
        scores_T = _tc_scores(x, W, c)
        outs.append(_sc_top2(scores_T))
    i1 = jnp.concatenate([o[0] for o in outs])
    i2 = jnp.concatenate([o[1] for o in outs])
    v1 = jnp.concatenate([o[2] for o in outs])
    v2 = jnp.concatenate([o[3] for o in outs])
    topi = jnp.stack([i1, i2], axis=1)
    topv = jnp.stack([v1, v2], axis=1)
    return (topi, topv)


# speedup vs baseline: 1.0034x; 1.0034x over previous
import functools

import jax
import jax.numpy as jnp
from jax import lax
from jax.experimental import pallas as pl
from jax.experimental.pallas import tpu as pltpu
from jax.experimental.pallas import tpu_sc as plsc

TOKENS = 16384
D_MODEL = 4096
N_EXPERTS = 64
BT = 1024
NK = 2
KS = D_MODEL // NK

N_CHUNKS = 1
TOK_C = TOKENS // N_CHUNKS

NC = 2
NS = 16
NW = NC * NS
L = 16
CHUNK = TOK_C // NW
NG = CHUNK // L


WPB = BT // (TOKENS // NW)
HALF = (TOKENS // NW) // 2


def _scores_block(*refs):
    x_refs = refs[:NK]
    w_ref = refs[NK]
    out_ref = refs[NK + 1]
    acc = jnp.zeros((N_EXPERTS, BT), jnp.float32)
    for j in range(NK):
        acc += lax.dot_general(
            w_ref[:, j * KS:(j + 1) * KS], x_refs[j][...],
            dimension_numbers=(((1,), (1,)), ((), ())),
            preferred_element_type=jnp.float32,
        )
    for a in range(WPB):
        for b in range(2):
            s = (a * 2 + b) * HALF
            out_ref[a, b] = acc[:, s:s + HALF]


def _tc_scores(x, W, c):
    grid = (TOK_C // BT,)
    t0 = c * (TOK_C // BT)
    return pl.pallas_call(
        _scores_block,
        grid=grid,
        in_specs=[
            pl.BlockSpec((BT, KS),
                         functools.partial(lambda j, t: (t + t0, j), j))
            for j in range(NK)
        ] + [
            pl.BlockSpec((N_EXPERTS, D_MODEL), lambda t: (0, 0)),
        ],
        out_specs=pl.BlockSpec((WPB, 2, N_EXPERTS, HALF),
                               lambda t: (t, 0, 0, 0)),
        out_shape=jax.ShapeDtypeStruct((NW, 2, N_EXPERTS, HALF),
                                       jnp.float32),
    )(*([x] * NK), W)


def _top2_update(v, e, st):
    m1, i1, m2, i2 = st
    ev = jnp.full((L,), e, jnp.int32)
    gt1 = v > m1
    gt2 = v > m2
    m2n = jnp.where(gt1, m1, jnp.where(gt2, v, m2))
    i2n = jnp.where(gt1, i1, jnp.where(gt2, ev, i2))
    m1n = jnp.where(gt1, v, m1)
    i1n = jnp.where(gt1, ev, i1)
    return (m1n, i1n, m2n, i2n)


NI = 4


def _sc_top2_body(scores_hbm, i1_hbm, i2_hbm, v1_hbm, v2_hbm,
                  sbuf_a, sbuf_b, i1b, i2b, v1b, v2b, sem_a, sem_b, sem_o):
    wid = lax.axis_index("s") * NC + lax.axis_index("c")
    base = wid * CHUNK
    cp_a = pltpu.async_copy(scores_hbm.at[wid, 0], sbuf_a, sem_a)
    cp_b = pltpu.async_copy(scores_hbm.at[wid, 1], sbuf_b, sem_b)

    def make_quad(sbuf, out_base):
        def quad(g, _):
            offs = [g * (NI * L) + k * L for k in range(NI)]
            neg = jnp.full((L,), -jnp.inf, jnp.float32)
            zero = jnp.zeros((L,), jnp.int32)
            sts = [(neg, zero, neg, zero)] * NI
            for e in range(N_EXPERTS):
                sts = [_top2_update(sbuf[e, pl.ds(off, L)], e, st)
                       for off, st in zip(offs, sts)]
            for off, (m1, i1, m2, i2) in zip(offs, sts):
                e2 = jnp.exp(m2 - m1)
                d = 1.0 + e2
                o = out_base + off
                i1b[pl.ds(o, L)] = i1
                i2b[pl.ds(o, L)] = i2
                v1b[pl.ds(o, L)] = 1.0 / d
                v2b[pl.ds(o, L)] = e2 / d
            return 0
        return quad

    half_quads = HALF // (NI * L)
    cp_a.wait()
    lax.fori_loop(0, half_quads, make_quad(sbuf_a, 0), 0)
    cps_a = [pltpu.async_copy(b.at[pl.ds(0, HALF)],
                              h.at[pl.ds(base, HALF)], sem_o)
             for b, h in ((i1b, i1_hbm), (i2b, i2_hbm),
                          (v1b, v1_hbm), (v2b, v2_hbm))]
    cp_b.wait()
    lax.fori_loop(0, half_quads, make_quad(sbuf_b, HALF), 0)
    cps_b = [pltpu.async_copy(b.at[pl.ds(HALF, HALF)],
                              h.at[pl.ds(base + HALF, HALF)], sem_o)
             for b, h in ((i1b, i1_hbm), (i2b, i2_hbm),
                          (v1b, v1_hbm), (v2b, v2_hbm))]
    for cp in cps_a + cps_b:
        cp.wait()


def _sc_top2(scores_T):
    mesh = plsc.VectorSubcoreMesh(core_axis_name="c", subcore_axis_name="s")
    f = functools.partial(
        pl.kernel, mesh=mesh,
        out_type=[
            jax.ShapeDtypeStruct((TOK_C,), jnp.int32),
            jax.ShapeDtypeStruct((TOK_C,), jnp.int32),
            jax.ShapeDtypeStruct((TOK_C,), jnp.float32),
            jax.ShapeDtypeStruct((TOK_C,), jnp.float32),
        ],
        scratch_types=[
            pltpu.VMEM((N_EXPERTS, HALF), jnp.float32),
            pltpu.VMEM((N_EXPERTS, HALF), jnp.float32),
            pltpu.VMEM((CHUNK,), jnp.int32),
            pltpu.VMEM((CHUNK,), jnp.int32),
            pltpu.VMEM((CHUNK,), jnp.float32),
            pltpu.VMEM((CHUNK,), jnp.float32),
            pltpu.SemaphoreType.DMA,
            pltpu.SemaphoreType.DMA,
            pltpu.SemaphoreType.DMA,
        ],
    )(_sc_top2_body)
    return f(scores_T)


@jax.jit
def kernel(x, W):
    outs = []
    for c in range(N_CHUNKS):
        scores_T = _tc_scores(x, W, c)
        outs.append(_sc_top2(scores_T))
    i1 = jnp.concatenate([o[0] for o in outs])
    i2 = jnp.concatenate([o[1] for o in outs])
    v1 = jnp.concatenate([o[2] for o in outs])
    v2 = jnp.concatenate([o[3] for o in outs])
    topi = jnp.stack([i1, i2], axis=1)
    topv = jnp.stack([v1, v2], axis=1)
    return (topi, topv)
